# baseline (speedup 1.0000x reference)
import functools

import jax
import jax.numpy as jnp
from jax import lax
from jax.experimental import pallas as pl
from jax.experimental.pallas import tpu as pltpu
from jax.experimental.pallas import tpu_sc as plsc

B = 4096
NL = 10
CH = 6400
LANES = 16
NW = 32
ROWS_PER_W = B // NW
CHUNKS = CH // LANES


def _sc_partials_kernel(z_hbm, labels_hbm, a_out, q_out, c_out,
                        a_v, zbuf0, zbuf1, labels_v, q_v, c_v, sem0, sem1):
    nc = 2
    wid = lax.axis_index("s") * nc + lax.axis_index("c")
    base = wid * ROWS_PER_W

    zeros = jnp.zeros((LANES,), jnp.float32)
    ones = jnp.ones((LANES,), jnp.float32)

    pltpu.sync_copy(labels_hbm.at[pl.ds(base, ROWS_PER_W)], labels_v)

    def zero_body(i, c):
        a_v[pl.ds(LANES * i, LANES)] = zeros
        return c
    lax.fori_loop(0, NL * CHUNKS, zero_body, 0)
    for l in range(NL):
        q_v[pl.ds(LANES * l, LANES)] = zeros
        c_v[pl.ds(LANES * l, LANES)] = zeros

    bufs = (zbuf0, zbuf1)
    sems = (sem0, sem1)

    def start(k, row):
        pltpu.make_async_copy(z_hbm.at[base + row], bufs[k], sems[k]).start()

    def wait(k):
        pltpu.make_async_copy(z_hbm.at[base], bufs[k], sems[k]).wait()

    def process(buf, lab):
        off = lab * CH

        def body(j, q):
            for u in range(4):
                o = LANES * (4 * j + u)
                zv = buf[pl.ds(o, LANES)]
                plsc.addupdate(a_v.at[pl.ds(off + o, LANES)], zv)
                q = q + zv * zv
            return q
        q = lax.fori_loop(0, CHUNKS // 4, body, zeros)
        plsc.addupdate(q_v.at[pl.ds(lab * LANES, LANES)], q)
        plsc.addupdate(c_v.at[pl.ds(lab * LANES, LANES)], ones)

    NGROUPS = ROWS_PER_W // LANES
    start(0, 0)
    start(1, 1)

    def group_body(g, c):
        lv = labels_v[pl.ds(LANES * g, LANES)]
        for u in range(LANES):
            k = u % 2
            wait(k)
            process(bufs[k], lv[u])
            start(k, LANES * g + u + 2)
        return c
    lax.fori_loop(0, NGROUPS - 1, group_body, 0)
    lv = labels_v[pl.ds(LANES * (NGROUPS - 1), LANES)]
    for u in range(LANES):
        k = u % 2
        wait(k)
        process(bufs[k], lv[u])
        if u < LANES - 2:
            start(k, LANES * (NGROUPS - 1) + u + 2)

    pltpu.sync_copy(a_v, a_out.at[wid])
    pltpu.sync_copy(q_v, q_out.at[wid])
    pltpu.sync_copy(c_v, c_out.at[wid])


def _finalize_body(a_ref, q_ref, c_ref, out_ref):
    a = jnp.sum(a_ref[...], axis=0)
    q = jnp.sum(q_ref[...], axis=(0, 2))
    n = jnp.sum(c_ref[...][:, :, 0], axis=0)
    ssq = jnp.sum(a * a, axis=1)
    safe = jnp.maximum(n, 1.0)
    chf = jnp.float32(CH)
    eps2 = jnp.float32(1e-16)
    mse = q / (safe * chf) - ssq / (safe * safe * chf) + eps2
    out_ref[...] = jnp.sum(jnp.where(n > 0, mse, 0.0)).reshape(1, 1)


@jax.jit
def _run(z2d, labels):
    mesh = plsc.VectorSubcoreMesh(core_axis_name="c", subcore_axis_name="s")
    sc = pl.kernel(
        _sc_partials_kernel,
        mesh=mesh,
        out_type=(
            jax.ShapeDtypeStruct((NW, NL * CH), jnp.float32),
            jax.ShapeDtypeStruct((NW, NL * LANES), jnp.float32),
            jax.ShapeDtypeStruct((NW, NL * LANES), jnp.float32),
        ),
        scratch_types=[
            pltpu.VMEM((NL * CH,), jnp.float32),
            pltpu.VMEM((CH,), jnp.float32),
            pltpu.VMEM((CH,), jnp.float32),
            pltpu.VMEM((ROWS_PER_W,), jnp.int32),
            pltpu.VMEM((NL * LANES,), jnp.float32),
            pltpu.VMEM((NL * LANES,), jnp.float32),
            pltpu.SemaphoreType.DMA,
            pltpu.SemaphoreType.DMA,
        ],
    )
    a_part, q_part, c_part = sc(z2d, labels)

    out = pl.pallas_call(
        _finalize_body,
        out_shape=jax.ShapeDtypeStruct((1, 1), jnp.float32),
    )(a_part.reshape(NW, NL, CH),
      q_part.reshape(NW, NL, LANES),
      c_part.reshape(NW, NL, LANES))
    return out[0, 0]


def kernel(z, labels):
    return _run(z.reshape(B, CH), labels)
